# baseline (speedup 1.0000x reference)
import jax
import jax.numpy as jnp
from jax import lax
from jax.experimental import pallas as pl
from jax.experimental.pallas import tpu as pltpu
from jax.experimental.pallas import tpu_sc as plsc

_NUM_CORES = 2
_NUM_SUBCORES = 16
_NW = _NUM_CORES * _NUM_SUBCORES
_CHUNK = 65024


def _sc_copy_body(words_per_w):
    nfull = words_per_w // _CHUNK
    tail = words_per_w - nfull * _CHUNK
    sizes = [_CHUNK] * nfull + ([tail] if tail else [])
    starts = [i * _CHUNK for i in range(len(sizes))]
    nchunks = len(sizes)

    def body(x_hbm, o_hbm, buf0, buf1, isem0, isem1, osem0, osem1):
        bufs = (buf0, buf1)
        isems = (isem0, isem1)
        osems = (osem0, osem1)
        c = lax.axis_index("c")
        s = lax.axis_index("s")
        base = (s * _NUM_CORES + c) * words_per_w

        def in_copy(i):
            b = i % 2
            return pltpu.make_async_copy(
                x_hbm.at[pl.ds(base + starts[i], sizes[i])],
                bufs[b].at[pl.ds(0, sizes[i])], isems[b])

        def out_copy(i):
            b = i % 2
            return pltpu.make_async_copy(
                bufs[b].at[pl.ds(0, sizes[i])],
                o_hbm.at[pl.ds(base + starts[i], sizes[i])], osems[b])

        in_copy(0).start()
        for i in range(nchunks):
            if i + 1 < nchunks:
                if i >= 1:
                    out_copy(i - 1).wait()
                in_copy(i + 1).start()
            in_copy(i).wait()
            out_copy(i).start()
        if nchunks >= 2:
            out_copy(nchunks - 2).wait()
        out_copy(nchunks - 1).wait()

    return body


def kernel(data, partitions):
    del partitions
    n, d = data.shape
    total = n * d
    words_per_w = total // _NW
    mesh = plsc.VectorSubcoreMesh(
        core_axis_name="c", subcore_axis_name="s",
        num_cores=_NUM_CORES, num_subcores=_NUM_SUBCORES)
    sc_copy = pl.kernel(
        _sc_copy_body(words_per_w),
        out_type=jax.ShapeDtypeStruct((total,), data.dtype),
        mesh=mesh,
        scratch_types=(
            [pltpu.VMEM((_CHUNK,), jnp.float32)] * 2
            + [pltpu.SemaphoreType.DMA] * 4),
    )
    return sc_copy(data.reshape(total)).reshape(n, d)


# speedup vs baseline: 1.3638x; 1.3638x over previous
import jax
import jax.numpy as jnp
from jax import lax
from jax.experimental import pallas as pl
from jax.experimental.pallas import tpu as pltpu

_CHUNK_ROWS = 8192
_NSLOTS = 8


def _make_copy_kernel(nchunks):
    ngroups = nchunks // _NSLOTS

    def _copy(x_hbm, o_hbm, buf, *sems):
        in_sems, out_sems = sems[:_NSLOTS], sems[_NSLOTS:]

        def in_copy(i, b):
            return pltpu.make_async_copy(
                x_hbm.at[pl.ds(i * _CHUNK_ROWS, _CHUNK_ROWS)],
                buf.at[b], in_sems[b])

        def out_copy(i, b):
            return pltpu.make_async_copy(
                buf.at[b],
                o_hbm.at[pl.ds(i * _CHUNK_ROWS, _CHUNK_ROWS)], out_sems[b])

        for b in range(_NSLOTS):
            in_copy(b, b).start()

        def body(g, carry):
            i0 = g * _NSLOTS
            for b in range(_NSLOTS):
                in_copy(i0 + b, b).wait()
                out_copy(i0 + b, b).start()
            for b in range(_NSLOTS):
                out_copy(i0 + b, b).wait()

                @pl.when(i0 + b + _NSLOTS < nchunks)
                def _():
                    in_copy(i0 + b + _NSLOTS, b).start()

            return carry

        lax.fori_loop(0, ngroups, body, 0)

    return _copy


def kernel(data, partitions):
    del partitions
    n, d = data.shape
    nchunks = n // _CHUNK_ROWS
    return pl.pallas_call(
        _make_copy_kernel(nchunks),
        in_specs=[pl.BlockSpec(memory_space=pl.ANY)],
        out_specs=pl.BlockSpec(memory_space=pl.ANY),
        out_shape=jax.ShapeDtypeStruct((n, d), data.dtype),
        scratch_shapes=(
            [pltpu.VMEM((_NSLOTS, _CHUNK_ROWS, d), jnp.float32)]
            + [pltpu.SemaphoreType.DMA] * (2 * _NSLOTS)),
    )(data)
